# baseline (device time: 46508 ns/iter reference)
import jax
import jax.numpy as jnp
from jax import lax
from jax.experimental import pallas as pl
from jax.experimental.pallas import tpu as pltpu

N_DEV = 8
GENS = (1, 3, 4)


def kernel(A, B):
    m, k = A.shape
    _, n = B.shape
    f32 = jnp.float32
    bf16 = jnp.bfloat16
    third = m // 3
    R = third // 2
    h0 = R // 2
    h1 = R // 4
    insts = [(t, u) for u in range(2) for t in range(3)]
    jidx = {(t, u): t * 2 + u for t, u in insts}

    def body(a_ref, b_ref, out_ref, a16, b16, sb0, sb1, sb2,
             rb0, rb1, rb2, ag_buf, rs_send, rs_recv, ag_send, ag_recv):
        me = lax.axis_index("i")
        bit0 = me & 1
        bit1 = (me >> 1) & 1
        bit2 = (me >> 2) & 1
        c = (bit0 ^ bit1, bit1, bit2)

        barrier = pltpu.get_barrier_semaphore()
        for g in GENS:
            pl.semaphore_signal(
                barrier, inc=1,
                device_id=(me ^ g,), device_id_type=pl.DeviceIdType.MESH,
            )
        pl.semaphore_wait(barrier, 3)

        a16[:, :] = a_ref[:, :].astype(bf16)
        b16[:, :] = b_ref[:, :].astype(bf16)

        sbufs = [sb0, sb1, sb2]
        rbufs = [rb0, rb1, rb2]

        def mm(r0, rows):
            return jnp.dot(a16[pl.ds(r0, rows), :], b16[:, :],
                           preferred_element_type=f32)

        def make_rs(t, u, s):
            j = jidx[(t, u)]
            return pltpu.make_async_remote_copy(
                src_ref=sbufs[s].at[j],
                dst_ref=rbufs[s].at[j],
                send_sem=rs_send.at[j, s],
                recv_sem=rs_recv.at[j, s],
                device_id=(me ^ GENS[(t + s) % 3],),
                device_id_type=pl.DeviceIdType.MESH,
            )

        starts = {}
        rdmas = {}
        for t, u in insts:
            j = jidx[(t, u)]
            base = t * third + u * R
            ck = c[t]
            send = base + (1 - ck) * h0
            sbufs[0][j, :, :] = mm(send, h0).astype(bf16)
            rdmas[(t, u)] = make_rs(t, u, 0)
            rdmas[(t, u)].start()
            starts[(t, u)] = base + ck * h0
        for t, u in insts:
            out_ref[pl.ds(starts[(t, u)], h0), :] = mm(starts[(t, u)], h0)

        done = [rdmas[i] for i in insts]
        new_rdmas = {}
        acc_keep = {}
        for t, u in insts:
            j = jidx[(t, u)]
            rdmas[(t, u)].wait_recv()
            ck = c[(t + 1) % 3]
            keep = starts[(t, u)] + ck * h1
            send = starts[(t, u)] + (1 - ck) * h1
            acc_send = (out_ref[pl.ds(send, h1), :]
                        + rbufs[0][j, pl.ds((1 - ck) * h1, h1), :].astype(f32))
            sbufs[1][j, :, :] = acc_send.astype(bf16)
            new_rdmas[(t, u)] = make_rs(t, u, 1)
            new_rdmas[(t, u)].start()
            acc_keep[(t, u)] = (
                out_ref[pl.ds(keep, h1), :]
                + rbufs[0][j, pl.ds(ck * h1, h1), :].astype(f32)
            )
            starts[(t, u)] = keep
        rdmas = new_rdmas

        done += [rdmas[i] for i in insts]
        new_rdmas = {}
        acc_full = {}
        for t, u in insts:
            j = jidx[(t, u)]
            rdmas[(t, u)].wait_recv()
            acc = acc_keep[(t, u)] + rbufs[1][j].astype(f32)
            sbufs[2][j, :, :] = acc.astype(bf16)
            new_rdmas[(t, u)] = make_rs(t, u, 2)
            new_rdmas[(t, u)].start()
            acc_full[(t, u)] = acc
        rdmas = new_rdmas

        done += [rdmas[i] for i in insts]
        ag0 = {}
        for t, u in insts:
            j = jidx[(t, u)]
            rdmas[(t, u)].wait_recv()
            z = acc_full[(t, u)] + rbufs[2][j].astype(f32)
            g = 0.5 * z * (1.0 + jnp.tanh(
                0.7978845608 * (z + 0.044715 * z * z * z)))
            ag_buf[pl.ds(starts[(t, u)], h1), :] = g.astype(bf16)
            ag0[(t, u)] = pltpu.make_async_remote_copy(
                src_ref=ag_buf.at[pl.ds(starts[(t, u)], h1)],
                dst_ref=ag_buf.at[pl.ds(starts[(t, u)], h1)],
                send_sem=ag_send.at[j, 0],
                recv_sem=ag_recv.at[j, 0],
                device_id=(me ^ GENS[(t + 1) % 3],),
                device_id_type=pl.DeviceIdType.MESH,
            )
            ag0[(t, u)].start()
            out_ref[pl.ds(starts[(t, u)], h1), :] = g

        done += [ag0[i] for i in insts]
        ag1 = {}
        pending = []
        for t, u in insts:
            j = jidx[(t, u)]
            ag0[(t, u)].wait_recv()
            ck = c[(t + 1) % 3]
            new_start = starts[(t, u)] - ck * h1
            pending.append((new_start + (1 - ck) * h1, h1))
            starts[(t, u)] = new_start
            ag1[(t, u)] = pltpu.make_async_remote_copy(
                src_ref=ag_buf.at[pl.ds(new_start, h0)],
                dst_ref=ag_buf.at[pl.ds(new_start, h0)],
                send_sem=ag_send.at[j, 1],
                recv_sem=ag_recv.at[j, 1],
                device_id=(me ^ GENS[t],),
                device_id_type=pl.DeviceIdType.MESH,
            )
            ag1[(t, u)].start()
        for r0, rows in pending:
            out_ref[pl.ds(r0, rows), :] = ag_buf[pl.ds(r0, rows), :].astype(f32)
        pending = []
        for t, u in insts:
            ag1[(t, u)].wait_recv()
            ck = c[t]
            new_start = starts[(t, u)] - ck * h0
            pending.append((new_start + (1 - ck) * h0, h0))
            starts[(t, u)] = new_start
        for r0, rows in pending:
            out_ref[pl.ds(r0, rows), :] = ag_buf[pl.ds(r0, rows), :].astype(f32)

        done += [ag1[i] for i in insts]
        for rdma in done:
            rdma.wait_send()

    return pl.pallas_call(
        body,
        out_shape=jax.ShapeDtypeStruct((m, n), f32),
        in_specs=[
            pl.BlockSpec(memory_space=pltpu.VMEM),
            pl.BlockSpec(memory_space=pltpu.VMEM),
        ],
        out_specs=pl.BlockSpec(memory_space=pltpu.VMEM),
        scratch_shapes=[
            pltpu.VMEM((m, k), bf16),
            pltpu.VMEM((k, n), bf16),
            pltpu.VMEM((6, h0, n), bf16),
            pltpu.VMEM((6, h1, n), bf16),
            pltpu.VMEM((6, h1, n), bf16),
            pltpu.VMEM((6, h0, n), bf16),
            pltpu.VMEM((6, h1, n), bf16),
            pltpu.VMEM((6, h1, n), bf16),
            pltpu.VMEM((m, n), bf16),
            pltpu.SemaphoreType.DMA((6, 3)),
            pltpu.SemaphoreType.DMA((6, 3)),
            pltpu.SemaphoreType.DMA((6, 2)),
            pltpu.SemaphoreType.DMA((6, 2)),
        ],
        compiler_params=pltpu.CompilerParams(collective_id=0),
    )(A, B)


# device time: 46228 ns/iter; 1.0061x vs baseline; 1.0061x over previous
import jax
import jax.numpy as jnp
from jax import lax
from jax.experimental import pallas as pl
from jax.experimental.pallas import tpu as pltpu

N_DEV = 8
GENS = (1, 3, 4)


def kernel(A, B):
    m, k = A.shape
    _, n = B.shape
    f32 = jnp.float32
    bf16 = jnp.bfloat16
    third = m // 3
    R = third // 2
    h0 = R // 2
    h1 = R // 4
    insts = [(t, u) for u in range(2) for t in range(3)]
    jidx = {(t, u): t * 2 + u for t, u in insts}

    def body(a_ref, b_ref, out_ref, sb0, sb1, sb2,
             rb0, rb1, rb2, ag_buf, rs_send, rs_recv, ag_send, ag_recv):
        me = lax.axis_index("i")
        bit0 = me & 1
        bit1 = (me >> 1) & 1
        bit2 = (me >> 2) & 1
        c = (bit0 ^ bit1, bit1, bit2)

        barrier = pltpu.get_barrier_semaphore()
        for g in GENS:
            pl.semaphore_signal(
                barrier, inc=1,
                device_id=(me ^ g,), device_id_type=pl.DeviceIdType.MESH,
            )
        pl.semaphore_wait(barrier, 3)

        sbufs = [sb0, sb1, sb2]
        rbufs = [rb0, rb1, rb2]

        def mm(r0, rows):
            return jnp.dot(a_ref[pl.ds(r0, rows), :], b_ref[:, :],
                           preferred_element_type=f32)

        def make_rs(t, u, s):
            j = jidx[(t, u)]
            return pltpu.make_async_remote_copy(
                src_ref=sbufs[s].at[j],
                dst_ref=rbufs[s].at[j],
                send_sem=rs_send.at[j, s],
                recv_sem=rs_recv.at[j, s],
                device_id=(me ^ GENS[(t + s) % 3],),
                device_id_type=pl.DeviceIdType.MESH,
            )

        starts = {}
        rdmas = {}
        for t, u in insts:
            j = jidx[(t, u)]
            base = t * third + u * R
            ck = c[t]
            send = base + (1 - ck) * h0
            sbufs[0][j, :, :] = mm(send, h0).astype(bf16)
            rdmas[(t, u)] = make_rs(t, u, 0)
            rdmas[(t, u)].start()
            starts[(t, u)] = base + ck * h0
        for t, u in insts:
            out_ref[pl.ds(starts[(t, u)], h0), :] = mm(starts[(t, u)], h0)

        done = [rdmas[i] for i in insts]
        new_rdmas = {}
        acc_keep = {}
        for t, u in insts:
            j = jidx[(t, u)]
            rdmas[(t, u)].wait_recv()
            ck = c[(t + 1) % 3]
            keep = starts[(t, u)] + ck * h1
            send = starts[(t, u)] + (1 - ck) * h1
            acc_send = (out_ref[pl.ds(send, h1), :]
                        + rbufs[0][j, pl.ds((1 - ck) * h1, h1), :].astype(f32))
            sbufs[1][j, :, :] = acc_send.astype(bf16)
            new_rdmas[(t, u)] = make_rs(t, u, 1)
            new_rdmas[(t, u)].start()
            acc_keep[(t, u)] = (
                out_ref[pl.ds(keep, h1), :]
                + rbufs[0][j, pl.ds(ck * h1, h1), :].astype(f32)
            )
            starts[(t, u)] = keep
        rdmas = new_rdmas

        done += [rdmas[i] for i in insts]
        new_rdmas = {}
        acc_full = {}
        for t, u in insts:
            j = jidx[(t, u)]
            rdmas[(t, u)].wait_recv()
            acc = acc_keep[(t, u)] + rbufs[1][j].astype(f32)
            sbufs[2][j, :, :] = acc.astype(bf16)
            new_rdmas[(t, u)] = make_rs(t, u, 2)
            new_rdmas[(t, u)].start()
            acc_full[(t, u)] = acc
        rdmas = new_rdmas

        done += [rdmas[i] for i in insts]
        ag0 = {}
        for t, u in insts:
            j = jidx[(t, u)]
            rdmas[(t, u)].wait_recv()
            z = acc_full[(t, u)] + rbufs[2][j].astype(f32)
            g = 0.5 * z * (1.0 + jnp.tanh(
                0.7978845608 * (z + 0.044715 * z * z * z)))
            ag_buf[pl.ds(starts[(t, u)], h1), :] = g.astype(bf16)
            ag0[(t, u)] = pltpu.make_async_remote_copy(
                src_ref=ag_buf.at[pl.ds(starts[(t, u)], h1)],
                dst_ref=ag_buf.at[pl.ds(starts[(t, u)], h1)],
                send_sem=ag_send.at[j, 0],
                recv_sem=ag_recv.at[j, 0],
                device_id=(me ^ GENS[(t + 1) % 3],),
                device_id_type=pl.DeviceIdType.MESH,
            )
            ag0[(t, u)].start()
            out_ref[pl.ds(starts[(t, u)], h1), :] = g

        done += [ag0[i] for i in insts]
        ag1 = {}
        pending = []
        for t, u in insts:
            j = jidx[(t, u)]
            ag0[(t, u)].wait_recv()
            ck = c[(t + 1) % 3]
            new_start = starts[(t, u)] - ck * h1
            pending.append((new_start + (1 - ck) * h1, h1))
            starts[(t, u)] = new_start
            own0 = starts[(t, u)]
            sub = []
            for q in range(2):
                r = pltpu.make_async_remote_copy(
                    src_ref=ag_buf.at[pl.ds(own0 + q * h1, h1)],
                    dst_ref=ag_buf.at[pl.ds(own0 + q * h1, h1)],
                    send_sem=ag_send.at[j, 1 + q],
                    recv_sem=ag_recv.at[j, 1 + q],
                    device_id=(me ^ GENS[t],),
                    device_id_type=pl.DeviceIdType.MESH,
                )
                r.start()
                sub.append(r)
            ag1[(t, u)] = sub
        for r0, rows in pending:
            out_ref[pl.ds(r0, rows), :] = ag_buf[pl.ds(r0, rows), :].astype(f32)
        for q in range(2):
            for t, u in insts:
                ag1[(t, u)][q].wait_recv()
                ck = c[t]
                p0 = (starts[(t, u)] - ck * h0) + (1 - ck) * h0 + q * h1
                out_ref[pl.ds(p0, h1), :] = ag_buf[pl.ds(p0, h1), :].astype(f32)

        done += [r for i in insts for r in ag1[i]]
        for rdma in done:
            rdma.wait_send()

    return pl.pallas_call(
        body,
        out_shape=jax.ShapeDtypeStruct((m, n), f32),
        in_specs=[
            pl.BlockSpec(memory_space=pltpu.VMEM),
            pl.BlockSpec(memory_space=pltpu.VMEM),
        ],
        out_specs=pl.BlockSpec(memory_space=pltpu.VMEM),
        scratch_shapes=[
            pltpu.VMEM((6, h0, n), bf16),
            pltpu.VMEM((6, h1, n), bf16),
            pltpu.VMEM((6, h1, n), bf16),
            pltpu.VMEM((6, h0, n), bf16),
            pltpu.VMEM((6, h1, n), bf16),
            pltpu.VMEM((6, h1, n), bf16),
            pltpu.VMEM((m, n), bf16),
            pltpu.SemaphoreType.DMA((6, 3)),
            pltpu.SemaphoreType.DMA((6, 3)),
            pltpu.SemaphoreType.DMA((6, 3)),
            pltpu.SemaphoreType.DMA((6, 3)),
        ],
        compiler_params=pltpu.CompilerParams(collective_id=0),
    )(A, B)


# device time: 45830 ns/iter; 1.0148x vs baseline; 1.0087x over previous
import jax
import jax.numpy as jnp
from jax import lax
from jax.experimental import pallas as pl
from jax.experimental.pallas import tpu as pltpu

N_DEV = 8
GENS = (1, 3, 4)


def kernel(A, B):
    m, k = A.shape
    _, n = B.shape
    f32 = jnp.float32
    bf16 = jnp.bfloat16
    third = m // 3
    R = third // 2
    h0 = R // 2
    h1 = R // 4
    insts = [(t, u) for u in range(2) for t in range(3)]
    jidx = {(t, u): t * 2 + u for t, u in insts}

    def body(a_ref, b_ref, out_ref, sb0, sb1, sb2,
             rb0, rb1, rb2, ag_buf, rs_send, rs_recv, ag_send, ag_recv):
        me = lax.axis_index("i")
        bit0 = me & 1
        bit1 = (me >> 1) & 1
        bit2 = (me >> 2) & 1
        c = (bit0 ^ bit1, bit1, bit2)

        barrier = pltpu.get_barrier_semaphore()
        for g in GENS:
            pl.semaphore_signal(
                barrier, inc=1,
                device_id=(me ^ g,), device_id_type=pl.DeviceIdType.MESH,
            )
        pl.semaphore_wait(barrier, 3)

        sbufs = [sb0, sb1, sb2]
        rbufs = [rb0, rb1, rb2]

        def mm(r0, rows):
            return jnp.dot(a_ref[pl.ds(r0, rows), :], b_ref[:, :],
                           preferred_element_type=f32)

        def make_rs(t, u, s):
            j = jidx[(t, u)]
            return pltpu.make_async_remote_copy(
                src_ref=sbufs[s].at[j],
                dst_ref=rbufs[s].at[j],
                send_sem=rs_send.at[j, s],
                recv_sem=rs_recv.at[j, s],
                device_id=(me ^ GENS[(t + s) % 3],),
                device_id_type=pl.DeviceIdType.MESH,
            )

        starts = {}
        rdmas = {}
        for t, u in insts:
            j = jidx[(t, u)]
            base = t * third + u * R
            ck = c[t]
            send = base + (1 - ck) * h0
            sbufs[0][j, :, :] = mm(send, h0).astype(bf16)
            rdmas[(t, u)] = make_rs(t, u, 0)
            rdmas[(t, u)].start()
            starts[(t, u)] = base + ck * h0
        for t, u in insts:
            out_ref[pl.ds(starts[(t, u)], h0), :] = mm(starts[(t, u)], h0)

        done = [rdmas[i] for i in insts]
        new_rdmas = {}
        acc_keep = {}
        for t, u in insts:
            j = jidx[(t, u)]
            rdmas[(t, u)].wait_recv()
            ck = c[(t + 1) % 3]
            keep = starts[(t, u)] + ck * h1
            send = starts[(t, u)] + (1 - ck) * h1
            acc_send = (out_ref[pl.ds(send, h1), :]
                        + rbufs[0][j, pl.ds((1 - ck) * h1, h1), :].astype(f32))
            sbufs[1][j, :, :] = acc_send.astype(bf16)
            new_rdmas[(t, u)] = make_rs(t, u, 1)
            new_rdmas[(t, u)].start()
            acc_keep[(t, u)] = (
                out_ref[pl.ds(keep, h1), :]
                + rbufs[0][j, pl.ds(ck * h1, h1), :].astype(f32)
            )
            starts[(t, u)] = keep
        rdmas = new_rdmas

        done += [rdmas[i] for i in insts]
        new_rdmas = {}
        acc_full = {}
        for t, u in insts:
            j = jidx[(t, u)]
            rdmas[(t, u)].wait_recv()
            acc = acc_keep[(t, u)] + rbufs[1][j].astype(f32)
            sbufs[2][j, :, :] = acc.astype(bf16)
            new_rdmas[(t, u)] = make_rs(t, u, 2)
            new_rdmas[(t, u)].start()
            acc_full[(t, u)] = acc
        rdmas = new_rdmas

        done += [rdmas[i] for i in insts]
        ag0 = {}
        ag1 = {}
        for t, u in insts:
            j = jidx[(t, u)]
            rdmas[(t, u)].wait_recv()
            z = acc_full[(t, u)] + rbufs[2][j].astype(f32)
            g = 0.5 * z * (1.0 + jnp.tanh(
                0.7978845608 * (z + 0.044715 * z * z * z)))
            ag_buf[pl.ds(starts[(t, u)], h1), :] = g.astype(bf16)
            ag0[(t, u)] = pltpu.make_async_remote_copy(
                src_ref=ag_buf.at[pl.ds(starts[(t, u)], h1)],
                dst_ref=ag_buf.at[pl.ds(starts[(t, u)], h1)],
                send_sem=ag_send.at[j, 0],
                recv_sem=ag_recv.at[j, 0],
                device_id=(me ^ GENS[(t + 1) % 3],),
                device_id_type=pl.DeviceIdType.MESH,
            )
            ag0[(t, u)].start()
            ck1 = c[(t + 1) % 3]
            base = starts[(t, u)] - ck1 * h1
            sub = []
            for q in range(2):
                r = pltpu.make_async_remote_copy(
                    src_ref=ag_buf.at[pl.ds(base + q * h1, h1)],
                    dst_ref=ag_buf.at[pl.ds(base + q * h1, h1)],
                    send_sem=ag_send.at[j, 1 + q],
                    recv_sem=ag_recv.at[j, 1 + q],
                    device_id=(me ^ GENS[t],),
                    device_id_type=pl.DeviceIdType.MESH,
                )
                sub.append(r)

                @pl.when(ck1 == q)
                def _(r=r):
                    r.start()

            ag1[(t, u)] = sub
            out_ref[pl.ds(starts[(t, u)], h1), :] = g

        done += [ag0[i] for i in insts]
        pending = []
        for t, u in insts:
            ag0[(t, u)].wait_recv()
            ck = c[(t + 1) % 3]
            new_start = starts[(t, u)] - ck * h1
            pending.append((new_start + (1 - ck) * h1, h1))
            starts[(t, u)] = new_start
            for q in range(2):
                @pl.when(ck == 1 - q)
                def _(r=ag1[(t, u)][q]):
                    r.start()

        for r0, rows in pending:
            out_ref[pl.ds(r0, rows), :] = ag_buf[pl.ds(r0, rows), :].astype(f32)
        for phase in range(2):
            for t, u in insts:
                ck1 = c[(t + 1) % 3]
                ckt = c[t]
                pbase = starts[(t, u)] + (1 - 2 * ckt) * h0
                for q in range(2):
                    cond = (ck1 == q) if phase == 0 else (ck1 == 1 - q)

                    @pl.when(cond)
                    def _(r=ag1[(t, u)][q], p0=pbase + q * h1):
                        r.wait_recv()
                        out_ref[pl.ds(p0, h1), :] = (
                            ag_buf[pl.ds(p0, h1), :].astype(f32))

        done += [r for i in insts for r in ag1[i]]
        for rdma in done:
            rdma.wait_send()

    return pl.pallas_call(
        body,
        out_shape=jax.ShapeDtypeStruct((m, n), f32),
        in_specs=[
            pl.BlockSpec(memory_space=pltpu.VMEM),
            pl.BlockSpec(memory_space=pltpu.VMEM),
        ],
        out_specs=pl.BlockSpec(memory_space=pltpu.VMEM),
        scratch_shapes=[
            pltpu.VMEM((6, h0, n), bf16),
            pltpu.VMEM((6, h1, n), bf16),
            pltpu.VMEM((6, h1, n), bf16),
            pltpu.VMEM((6, h0, n), bf16),
            pltpu.VMEM((6, h1, n), bf16),
            pltpu.VMEM((6, h1, n), bf16),
            pltpu.VMEM((m, n), bf16),
            pltpu.SemaphoreType.DMA((6, 3)),
            pltpu.SemaphoreType.DMA((6, 3)),
            pltpu.SemaphoreType.DMA((6, 3)),
            pltpu.SemaphoreType.DMA((6, 3)),
        ],
        compiler_params=pltpu.CompilerParams(collective_id=0),
    )(A, B)
